# baseline (speedup 1.0000x reference)
import functools

import jax
import jax.numpy as jnp
import numpy as np
from jax import lax
from jax.experimental import pallas as pl
from jax.experimental.pallas import tpu as pltpu
from jax.experimental.pallas import tpu_sc as plsc

N = 10000
E = 320000
D = 128
H = 256
YW = 144

NC = 2
NS = 16
NW = NC * NS
PER_W = E // NW
PWA = 4992
PWB = PER_W - PWA
EA = NW * PWA
EB = NW * PWB
CHUNK = 80
CHUNK3 = 64
NP = 10112
RPS = NP // NS
DRO = 80


def _sc_gather_body(pw, off0, feat_hbm, w_hbm, sidx_hbm, nidx_hbm,
                    ms_out, mn_out, wn_out,
                    sidx_all, nidx_all, wtab_v,
                    msA, mnA, wnA, msB, mnB, wnB,
                    gsemA, gsemB, wsemA, wsemB):
    nch = pw // CHUNK
    tail = pw - nch * CHUNK
    wid = lax.axis_index("s") * NC + lax.axis_index("c")
    ibase = wid * PER_W + off0
    obase = wid * pw
    pltpu.sync_copy(w_hbm, wtab_v)
    pltpu.sync_copy(sidx_hbm.at[pl.ds(ibase, pw)], sidx_all)
    pltpu.sync_copy(nidx_hbm.at[pl.ds(ibase, pw)], nidx_all)

    def fire_g(off, n, ms_v, mn_v, gsem):
        pltpu.async_copy(
            feat_hbm.at[sidx_all.at[pl.ds(off, n)]],
            ms_v.at[pl.ds(0, n)], gsem)
        pltpu.async_copy(
            feat_hbm.at[nidx_all.at[pl.ds(off, n)]],
            mn_v.at[pl.ds(0, n)], gsem)

    def wait_g(n, ms_v, mn_v, gsem):
        pltpu.make_async_copy(
            feat_hbm.at[pl.ds(0, n)], ms_v.at[pl.ds(0, n)], gsem).wait()
        pltpu.make_async_copy(
            feat_hbm.at[pl.ds(0, n)], mn_v.at[pl.ds(0, n)], gsem).wait()

    def wn_comp(off, n, wn_v):
        for k in range(n // 16):
            idx16 = nidx_all[pl.ds(off + k * 16, 16)]
            wn_v[pl.ds(k * 16, 16)] = plsc.load_gather(
                wtab_v, [lax.shift_right_logical(idx16, 7),
                         lax.bitwise_and(idx16, 127)])

    def fire_w(off, n, ms_v, mn_v, wn_v, wsem):
        base = obase + off
        pltpu.async_copy(ms_v.at[pl.ds(0, n)],
                         ms_out.at[pl.ds(base, n)], wsem)
        pltpu.async_copy(mn_v.at[pl.ds(0, n)],
                         mn_out.at[pl.ds(base, n)], wsem)
        pltpu.async_copy(wn_v.at[pl.ds(0, n)],
                         wn_out.at[pl.ds(base, n)], wsem)

    def wait_w(n, ms_v, mn_v, wn_v, wsem):
        pltpu.make_async_copy(
            ms_v.at[pl.ds(0, n)], ms_out.at[pl.ds(obase, n)], wsem).wait()
        pltpu.make_async_copy(
            mn_v.at[pl.ds(0, n)], mn_out.at[pl.ds(obase, n)], wsem).wait()
        pltpu.make_async_copy(
            wn_v.at[pl.ds(0, n)], wn_out.at[pl.ds(obase, n)], wsem).wait()

    fire_g(0, CHUNK, msA, mnA, gsemA)

    def body(j, _):
        ia = 2 * j
        ib = 2 * j + 1
        fire_g(ib * CHUNK, CHUNK, msB, mnB, gsemB)
        wait_g(CHUNK, msA, mnA, gsemA)
        wn_comp(ia * CHUNK, CHUNK, wnA)
        fire_w(ia * CHUNK, CHUNK, msA, mnA, wnA, wsemA)
        wait_w(CHUNK, msA, mnA, wnA, wsemA)

        @pl.when(ib + 1 < nch)
        def _():
            fire_g((ib + 1) * CHUNK, CHUNK, msA, mnA, gsemA)

        wait_g(CHUNK, msB, mnB, gsemB)
        wn_comp(ib * CHUNK, CHUNK, wnB)
        fire_w(ib * CHUNK, CHUNK, msB, mnB, wnB, wsemB)
        wait_w(CHUNK, msB, mnB, wnB, wsemB)
        return 0

    lax.fori_loop(0, nch // 2, body, 0)
    toff = nch * CHUNK
    fire_g(toff, tail, msA, mnA, gsemA)
    wait_g(tail, msA, mnA, gsemA)
    wn_comp(toff, tail, wnA)
    fire_w(toff, tail, msA, mnA, wnA, wsemA)
    wait_w(tail, msA, mnA, wnA, wsemA)


@functools.cache
def _make_sc_gather(pw, off0):
    mesh = plsc.VectorSubcoreMesh(core_axis_name="c", subcore_axis_name="s")
    ne = NW * pw
    return functools.partial(
        pl.kernel,
        out_type=(
            jax.ShapeDtypeStruct((ne, D), jnp.float32),
            jax.ShapeDtypeStruct((ne, D), jnp.float32),
            jax.ShapeDtypeStruct((ne,), jnp.float32),
        ),
        mesh=mesh,
        scratch_types=[
            pltpu.VMEM((pw,), jnp.int32),
            pltpu.VMEM((pw,), jnp.int32),
            pltpu.VMEM((DRO, 128), jnp.float32),
            pltpu.VMEM((CHUNK, D), jnp.float32),
            pltpu.VMEM((CHUNK, D), jnp.float32),
            pltpu.VMEM((CHUNK,), jnp.float32),
            pltpu.VMEM((CHUNK, D), jnp.float32),
            pltpu.VMEM((CHUNK, D), jnp.float32),
            pltpu.VMEM((CHUNK,), jnp.float32),
            pltpu.SemaphoreType.DMA,
            pltpu.SemaphoreType.DMA,
            pltpu.SemaphoreType.DMA,
            pltpu.SemaphoreType.DMA,
        ],
        compiler_params=pltpu.CompilerParams(needs_layout_passes=False),
    )(functools.partial(_sc_gather_body, pw, off0))


def _tc_mlp_body(ms_ref, mn_ref, wn_ref, w0_ref, b0_ref, w1_ref, b1_ref,
                 p_ref, cx_ref, cg_ref):
    msg = jnp.concatenate([ms_ref[...], mn_ref[...]], axis=1)
    h = jnp.dot(msg, w0_ref[...], preferred_element_type=jnp.float32)
    h = h + b0_ref[0:1, :]
    h = jnp.where(h >= 0, h, 0.01 * h)
    y = jnp.dot(h, w1_ref[...], preferred_element_type=jnp.float32)
    y = y + b1_ref[0:1, :]
    logw = jnp.log(wn_ref[...])
    p = p_ref[0, 0]
    g = jnp.exp(y[:, D:D + 1] + p * logw)
    cx_ref[...] = y[:, 0:D] * g
    cg_ref[...] = jnp.broadcast_to(g, (g.shape[0], 8))


def _tc_mlp(ms, mn, wn, w0, b0, w1, b1, p):
    ne = ms.shape[0]
    T = 1024
    grid = (pl.cdiv(ne, T),)
    return pl.pallas_call(
        _tc_mlp_body,
        grid=grid,
        in_specs=[
            pl.BlockSpec((T, D), lambda i: (i, 0)),
            pl.BlockSpec((T, D), lambda i: (i, 0)),
            pl.BlockSpec((T, 1), lambda i: (i, 0)),
            pl.BlockSpec((2 * D, 2 * H), lambda i: (0, 0)),
            pl.BlockSpec((8, 2 * H), lambda i: (0, 0)),
            pl.BlockSpec((2 * H, YW), lambda i: (0, 0)),
            pl.BlockSpec((8, YW), lambda i: (0, 0)),
            pl.BlockSpec((8, 128), lambda i: (0, 0)),
        ],
        out_specs=[
            pl.BlockSpec((T, D), lambda i: (i, 0)),
            pl.BlockSpec((T, 8), lambda i: (i, 0)),
        ],
        out_shape=(
            jax.ShapeDtypeStruct((ne, D), jnp.float32),
            jax.ShapeDtypeStruct((ne, 8), jnp.float32),
        ),
    )(ms, mn, wn, w0, b0, w1, b1, p)


def _sc_scatter_body(pw, off0, cx_hbm, cg_hbm, sidx_hbm, partx_out, partg_out,
                     sidxA, cxA, cgA, sidxB, cxB, cgB, sidxT, cxT, cgT,
                     den_v, rowid_v, acc, accg,
                     lsemA, lsemB, ssemA, ssemB):
    nch = pw // CHUNK3
    tail = pw - nch * CHUNK3
    cid = lax.axis_index("c")
    sid = lax.axis_index("s")
    wid = sid * NC + cid
    ibase = wid * PER_W + off0
    cbase = wid * pw

    def zden(j, _):
        def zcol(k, _):
            den_v[j, pl.ds(k * 16, 16)] = jnp.zeros((16,), jnp.float32)
            return 0
        lax.fori_loop(0, D // 16, zcol, 0)
        return 0

    lax.fori_loop(0, DRO, zden, 0)
    for k in range(DRO // 16):
        rowid_v[pl.ds(k * 16, 16)] = (
            lax.broadcasted_iota(jnp.int32, (16,), 0) + k * 16)

    @pl.when(sid == 0)
    def _():
        pltpu.sync_copy(den_v, accg)

    for z in range(RPS // DRO):
        pltpu.async_copy(
            den_v, acc.at[pl.ds(sid * RPS + z * DRO, DRO)], lsemA)
    rem = RPS % DRO
    if rem:
        pltpu.async_copy(
            den_v.at[pl.ds(0, rem)],
            acc.at[pl.ds(sid * RPS + (RPS // DRO) * DRO, rem)], lsemA)

    for z in range(RPS // DRO):
        pltpu.make_async_copy(
            den_v, acc.at[pl.ds(0, DRO)], lsemA).wait()
    if rem:
        pltpu.make_async_copy(
            den_v.at[pl.ds(0, rem)], acc.at[pl.ds(0, rem)], lsemA).wait()
    plsc.subcore_barrier()

    def fire_l(i, sidx_v, cx_v, cg_v, lsem):
        pltpu.async_copy(
            sidx_hbm.at[pl.ds(ibase + i * CHUNK3, CHUNK3)], sidx_v, lsem)
        pltpu.async_copy(
            cx_hbm.at[pl.ds(cbase + i * CHUNK3, CHUNK3)], cx_v, lsem)
        pltpu.async_copy(
            cg_hbm.at[pl.ds(cbase + i * CHUNK3, CHUNK3)], cg_v, lsem)

    def wait_l(sidx_v, cx_v, cg_v, lsem):
        pltpu.make_async_copy(
            sidx_hbm.at[pl.ds(0, CHUNK3)], sidx_v, lsem).wait()
        pltpu.make_async_copy(cx_hbm.at[pl.ds(0, CHUNK3)], cx_v, lsem).wait()
        pltpu.make_async_copy(cg_hbm.at[pl.ds(0, CHUNK3)], cg_v, lsem).wait()

    def den_comp(sidx_v, cg_v, nk):
        for k in range(nk):
            row16 = lax.broadcasted_iota(jnp.int32, (16,), 0) + k * 16
            col16 = jnp.zeros((16,), jnp.int32)
            g16 = plsc.load_gather(cg_v, [row16, col16])
            s16 = sidx_v[pl.ds(k * 16, 16)]
            plsc.addupdate_scatter(
                den_v, [lax.shift_right_logical(s16, 7),
                        lax.bitwise_and(s16, 127)], g16)

    def fire_s(sidx_v, cx_v, ssem):
        pltpu.async_copy(cx_v, acc.at[sidx_v], ssem, add=True)

    def wait_s(cx_v, ssem):
        pltpu.make_async_copy(cx_v, acc.at[pl.ds(0, CHUNK3)], ssem).wait()

    fire_l(0, sidxA, cxA, cgA, lsemA)

    def body(j, _):
        ib = 2 * j + 1
        fire_l(ib, sidxB, cxB, cgB, lsemB)
        wait_l(sidxA, cxA, cgA, lsemA)
        den_comp(sidxA, cgA, CHUNK3 // 16)
        fire_s(sidxA, cxA, ssemA)
        wait_s(cxA, ssemA)

        @pl.when(ib + 1 < nch)
        def _():
            fire_l(ib + 1, sidxA, cxA, cgA, lsemA)

        wait_l(sidxB, cxB, cgB, lsemB)
        den_comp(sidxB, cgB, CHUNK3 // 16)
        fire_s(sidxB, cxB, ssemB)
        wait_s(cxB, ssemB)
        return 0

    lax.fori_loop(0, nch // 2, body, 0)
    if tail:
        tb_i = ibase + nch * CHUNK3
        tb_c = cbase + nch * CHUNK3
        pltpu.sync_copy(sidx_hbm.at[pl.ds(tb_i, 16)], sidxT)
        pltpu.sync_copy(cx_hbm.at[pl.ds(tb_c, 16)], cxT)
        pltpu.sync_copy(cg_hbm.at[pl.ds(tb_c, 16)], cgT)
        den_comp(sidxT, cgT, 1)
        pltpu.sync_copy(cxT, acc.at[sidxT], add=True)
    plsc.subcore_barrier()
    pltpu.sync_copy(den_v, accg.at[rowid_v], add=True)
    plsc.subcore_barrier()

    row0 = sid * RPS
    pltpu.sync_copy(acc.at[pl.ds(row0, RPS)],
                    partx_out.at[pl.ds(cid * NP + row0, RPS)])

    @pl.when(sid == 0)
    def _():
        pltpu.sync_copy(accg, partg_out.at[pl.ds(cid * DRO, DRO)])


@functools.cache
def _make_sc_scatter(pw, off0):
    mesh = plsc.VectorSubcoreMesh(core_axis_name="c", subcore_axis_name="s")
    return functools.partial(
        pl.kernel,
        out_type=(
            jax.ShapeDtypeStruct((2 * NP, D), jnp.float32),
            jax.ShapeDtypeStruct((NC * DRO, 128), jnp.float32),
        ),
        mesh=mesh,
        scratch_types=[
            pltpu.VMEM((CHUNK3,), jnp.int32),
            pltpu.VMEM((CHUNK3, D), jnp.float32),
            pltpu.VMEM((CHUNK3, 8), jnp.float32),
            pltpu.VMEM((CHUNK3,), jnp.int32),
            pltpu.VMEM((CHUNK3, D), jnp.float32),
            pltpu.VMEM((CHUNK3, 8), jnp.float32),
            pltpu.VMEM((16,), jnp.int32),
            pltpu.VMEM((16, D), jnp.float32),
            pltpu.VMEM((16, 8), jnp.float32),
            pltpu.VMEM((DRO, 128), jnp.float32),
            pltpu.VMEM((DRO,), jnp.int32),
            pltpu.VMEM_SHARED((NP, D), jnp.float32),
            pltpu.VMEM_SHARED((DRO, 128), jnp.float32),
            pltpu.SemaphoreType.DMA,
            pltpu.SemaphoreType.DMA,
            pltpu.SemaphoreType.DMA,
            pltpu.SemaphoreType.DMA,
        ],
        compiler_params=pltpu.CompilerParams(needs_layout_passes=False),
    )(functools.partial(_sc_scatter_body, pw, off0))


def _tc_epi_body(x0_ref, x1_ref, x2_ref, x3_ref, den_ref, feat_ref, out_ref):
    num = (x0_ref[...] + x1_ref[...]) + (x2_ref[...] + x3_ref[...])
    out_ref[...] = num / (den_ref[...] + 1e-10) + feat_ref[...]


def _tc_epilogue(x0, x1, x2, x3, den, feat):
    T = 400
    grid = (N // T,)
    return pl.pallas_call(
        _tc_epi_body,
        grid=grid,
        in_specs=[
            pl.BlockSpec((T, D), lambda i: (i, 0)),
            pl.BlockSpec((T, D), lambda i: (i, 0)),
            pl.BlockSpec((T, D), lambda i: (i, 0)),
            pl.BlockSpec((T, D), lambda i: (i, 0)),
            pl.BlockSpec((T, 1), lambda i: (i, 0)),
            pl.BlockSpec((T, D), lambda i: (i, 0)),
        ],
        out_specs=pl.BlockSpec((T, D), lambda i: (i, 0)),
        out_shape=jax.ShapeDtypeStruct((N, D), jnp.float32),
    )(x0, x1, x2, x3, den, feat)


def kernel(node_weights, node_prev_features, self_idx, neighbor_idx, pow_p,
           gW0, gb0, gW1, gb1, mW0, mb0, mW1, mb1):
    feat = node_prev_features
    wpad = jnp.concatenate(
        [node_weights.reshape(N),
         jnp.zeros((DRO * 128 - N,), jnp.float32)]).reshape(DRO, 128)

    w0 = jnp.concatenate([gW0, mW0], axis=1)
    b0 = jnp.broadcast_to(jnp.concatenate([gb0, mb0])[None, :], (8, 2 * H))
    w1 = jnp.zeros((2 * H, YW), jnp.float32)
    w1 = w1.at[H:2 * H, 0:D].set(mW1)
    w1 = w1.at[0:H, D].set(gW1[:, 0])
    b1 = jnp.zeros((YW,), jnp.float32)
    b1 = b1.at[0:D].set(mb1)
    b1 = b1.at[D].set(gb1[0])
    b1 = jnp.broadcast_to(b1[None, :], (8, YW))
    p = jnp.broadcast_to(pow_p.reshape(1, 1), (8, 128))

    ms_a, mn_a, wn_a = _make_sc_gather(PWA, 0)(
        feat, wpad, self_idx, neighbor_idx)
    cx_a, cg_a = _tc_mlp(ms_a, mn_a, wn_a.reshape(EA, 1), w0, b0, w1, b1, p)
    ms_b, mn_b, wn_b = _make_sc_gather(PWB, PWA)(
        feat, wpad, self_idx, neighbor_idx)
    px_a, pg_a = _make_sc_scatter(PWA, 0)(cx_a, cg_a, self_idx)
    cx_b, cg_b = _tc_mlp(ms_b, mn_b, wn_b.reshape(EB, 1), w0, b0, w1, b1, p)
    px_b, pg_b = _make_sc_scatter(PWB, PWA)(cx_b, cg_b, self_idx)
    den = ((pg_a[0:DRO] + pg_a[DRO:2 * DRO])
           + (pg_b[0:DRO] + pg_b[DRO:2 * DRO])).reshape(DRO * 128)[0:N]
    return _tc_epilogue(px_a[0:N], px_a[NP:NP + N],
                        px_b[0:N], px_b[NP:NP + N],
                        den.reshape(N, 1), feat)


# speedup vs baseline: 1.0347x; 1.0347x over previous
import functools

import jax
import jax.numpy as jnp
import numpy as np
from jax import lax
from jax.experimental import pallas as pl
from jax.experimental.pallas import tpu as pltpu
from jax.experimental.pallas import tpu_sc as plsc

N = 10000
E = 320000
D = 128
H = 256
YW = 144

NC = 2
NS = 16
NW = NC * NS
PER_W = E // NW
PHASES = ((3328, 0), (3328, 3328), (3344, 6656))
CHUNK = 80
CHUNK3 = 64
NP = 10112
RPS = NP // NS
DRO = 80


def _sc_gather_body(pw, off0, feat_hbm, w_hbm, sidx_hbm, nidx_hbm,
                    ms_out, mn_out, wn_out,
                    sidx_all, nidx_all, wtab_v,
                    msA, mnA, wnA, msB, mnB, wnB,
                    gsemA, gsemB, wsemA, wsemB):
    nch = pw // CHUNK
    tail = pw - nch * CHUNK
    wid = lax.axis_index("s") * NC + lax.axis_index("c")
    ibase = wid * PER_W + off0
    obase = wid * pw
    pltpu.sync_copy(w_hbm, wtab_v)
    pltpu.sync_copy(sidx_hbm.at[pl.ds(ibase, pw)], sidx_all)
    pltpu.sync_copy(nidx_hbm.at[pl.ds(ibase, pw)], nidx_all)

    def fire_g(off, n, ms_v, mn_v, gsem):
        pltpu.async_copy(
            feat_hbm.at[sidx_all.at[pl.ds(off, n)]],
            ms_v.at[pl.ds(0, n)], gsem)
        pltpu.async_copy(
            feat_hbm.at[nidx_all.at[pl.ds(off, n)]],
            mn_v.at[pl.ds(0, n)], gsem)

    def wait_g(n, ms_v, mn_v, gsem):
        pltpu.make_async_copy(
            feat_hbm.at[pl.ds(0, n)], ms_v.at[pl.ds(0, n)], gsem).wait()
        pltpu.make_async_copy(
            feat_hbm.at[pl.ds(0, n)], mn_v.at[pl.ds(0, n)], gsem).wait()

    def wn_comp(off, n, wn_v):
        for k in range(n // 16):
            idx16 = nidx_all[pl.ds(off + k * 16, 16)]
            wn_v[pl.ds(k * 16, 16)] = plsc.load_gather(
                wtab_v, [lax.shift_right_logical(idx16, 7),
                         lax.bitwise_and(idx16, 127)])

    def fire_w(off, n, ms_v, mn_v, wn_v, wsem):
        base = obase + off
        pltpu.async_copy(ms_v.at[pl.ds(0, n)],
                         ms_out.at[pl.ds(base, n)], wsem)
        pltpu.async_copy(mn_v.at[pl.ds(0, n)],
                         mn_out.at[pl.ds(base, n)], wsem)
        pltpu.async_copy(wn_v.at[pl.ds(0, n)],
                         wn_out.at[pl.ds(base, n)], wsem)

    def wait_w(n, ms_v, mn_v, wn_v, wsem):
        pltpu.make_async_copy(
            ms_v.at[pl.ds(0, n)], ms_out.at[pl.ds(obase, n)], wsem).wait()
        pltpu.make_async_copy(
            mn_v.at[pl.ds(0, n)], mn_out.at[pl.ds(obase, n)], wsem).wait()
        pltpu.make_async_copy(
            wn_v.at[pl.ds(0, n)], wn_out.at[pl.ds(obase, n)], wsem).wait()

    fire_g(0, CHUNK, msA, mnA, gsemA)

    def body(j, _):
        ia = 2 * j
        ib = 2 * j + 1
        fire_g(ib * CHUNK, CHUNK, msB, mnB, gsemB)
        wait_g(CHUNK, msA, mnA, gsemA)
        wn_comp(ia * CHUNK, CHUNK, wnA)
        fire_w(ia * CHUNK, CHUNK, msA, mnA, wnA, wsemA)
        wait_w(CHUNK, msA, mnA, wnA, wsemA)

        @pl.when(ib + 1 < nch)
        def _():
            fire_g((ib + 1) * CHUNK, CHUNK, msA, mnA, gsemA)

        wait_g(CHUNK, msB, mnB, gsemB)
        wn_comp(ib * CHUNK, CHUNK, wnB)
        fire_w(ib * CHUNK, CHUNK, msB, mnB, wnB, wsemB)
        wait_w(CHUNK, msB, mnB, wnB, wsemB)
        return 0

    lax.fori_loop(0, nch // 2, body, 0)
    if nch % 2 == 1:
        wait_g(CHUNK, msA, mnA, gsemA)
        wn_comp((nch - 1) * CHUNK, CHUNK, wnA)
        fire_w((nch - 1) * CHUNK, CHUNK, msA, mnA, wnA, wsemA)
        wait_w(CHUNK, msA, mnA, wnA, wsemA)
    if tail:
        toff = nch * CHUNK
        fire_g(toff, tail, msA, mnA, gsemA)
        wait_g(tail, msA, mnA, gsemA)
        wn_comp(toff, tail, wnA)
        fire_w(toff, tail, msA, mnA, wnA, wsemA)
        wait_w(tail, msA, mnA, wnA, wsemA)


@functools.cache
def _make_sc_gather(pw, off0):
    mesh = plsc.VectorSubcoreMesh(core_axis_name="c", subcore_axis_name="s")
    ne = NW * pw
    return functools.partial(
        pl.kernel,
        out_type=(
            jax.ShapeDtypeStruct((ne, D), jnp.float32),
            jax.ShapeDtypeStruct((ne, D), jnp.float32),
            jax.ShapeDtypeStruct((ne,), jnp.float32),
        ),
        mesh=mesh,
        scratch_types=[
            pltpu.VMEM((pw,), jnp.int32),
            pltpu.VMEM((pw,), jnp.int32),
            pltpu.VMEM((DRO, 128), jnp.float32),
            pltpu.VMEM((CHUNK, D), jnp.float32),
            pltpu.VMEM((CHUNK, D), jnp.float32),
            pltpu.VMEM((CHUNK,), jnp.float32),
            pltpu.VMEM((CHUNK, D), jnp.float32),
            pltpu.VMEM((CHUNK, D), jnp.float32),
            pltpu.VMEM((CHUNK,), jnp.float32),
            pltpu.SemaphoreType.DMA,
            pltpu.SemaphoreType.DMA,
            pltpu.SemaphoreType.DMA,
            pltpu.SemaphoreType.DMA,
        ],
        compiler_params=pltpu.CompilerParams(needs_layout_passes=False),
    )(functools.partial(_sc_gather_body, pw, off0))


def _tc_mlp_body(ms_ref, mn_ref, wn_ref, w0_ref, b0_ref, w1_ref, b1_ref,
                 p_ref, cx_ref, cg_ref):
    msg = jnp.concatenate([ms_ref[...], mn_ref[...]], axis=1)
    h = jnp.dot(msg, w0_ref[...], preferred_element_type=jnp.float32)
    h = h + b0_ref[0:1, :]
    h = jnp.where(h >= 0, h, 0.01 * h)
    y = jnp.dot(h, w1_ref[...], preferred_element_type=jnp.float32)
    y = y + b1_ref[0:1, :]
    logw = jnp.log(wn_ref[...])
    p = p_ref[0, 0]
    g = jnp.exp(y[:, D:D + 1] + p * logw)
    cx_ref[...] = y[:, 0:D] * g
    cg_ref[...] = jnp.broadcast_to(g, (g.shape[0], 8))


def _tc_mlp(ms, mn, wn, w0, b0, w1, b1, p):
    ne = ms.shape[0]
    T = 1024
    grid = (pl.cdiv(ne, T),)
    return pl.pallas_call(
        _tc_mlp_body,
        grid=grid,
        in_specs=[
            pl.BlockSpec((T, D), lambda i: (i, 0)),
            pl.BlockSpec((T, D), lambda i: (i, 0)),
            pl.BlockSpec((T, 1), lambda i: (i, 0)),
            pl.BlockSpec((2 * D, 2 * H), lambda i: (0, 0)),
            pl.BlockSpec((8, 2 * H), lambda i: (0, 0)),
            pl.BlockSpec((2 * H, YW), lambda i: (0, 0)),
            pl.BlockSpec((8, YW), lambda i: (0, 0)),
            pl.BlockSpec((8, 128), lambda i: (0, 0)),
        ],
        out_specs=[
            pl.BlockSpec((T, D), lambda i: (i, 0)),
            pl.BlockSpec((T, 8), lambda i: (i, 0)),
        ],
        out_shape=(
            jax.ShapeDtypeStruct((ne, D), jnp.float32),
            jax.ShapeDtypeStruct((ne, 8), jnp.float32),
        ),
    )(ms, mn, wn, w0, b0, w1, b1, p)


def _sc_scatter_body(pw, off0, cx_hbm, cg_hbm, sidx_hbm, partx_out, partg_out,
                     sidxA, cxA, cgA, sidxB, cxB, cgB, sidxT, cxT, cgT,
                     den_v, rowid_v, acc, accg,
                     lsemA, lsemB, ssemA, ssemB):
    nch = pw // CHUNK3
    tail = pw - nch * CHUNK3
    cid = lax.axis_index("c")
    sid = lax.axis_index("s")
    wid = sid * NC + cid
    ibase = wid * PER_W + off0
    cbase = wid * pw

    def zden(j, _):
        def zcol(k, _):
            den_v[j, pl.ds(k * 16, 16)] = jnp.zeros((16,), jnp.float32)
            return 0
        lax.fori_loop(0, D // 16, zcol, 0)
        return 0

    lax.fori_loop(0, DRO, zden, 0)
    for k in range(DRO // 16):
        rowid_v[pl.ds(k * 16, 16)] = (
            lax.broadcasted_iota(jnp.int32, (16,), 0) + k * 16)

    @pl.when(sid == 0)
    def _():
        pltpu.sync_copy(den_v, accg)

    for z in range(RPS // DRO):
        pltpu.async_copy(
            den_v, acc.at[pl.ds(sid * RPS + z * DRO, DRO)], lsemA)
    rem = RPS % DRO
    if rem:
        pltpu.async_copy(
            den_v.at[pl.ds(0, rem)],
            acc.at[pl.ds(sid * RPS + (RPS // DRO) * DRO, rem)], lsemA)

    for z in range(RPS // DRO):
        pltpu.make_async_copy(
            den_v, acc.at[pl.ds(0, DRO)], lsemA).wait()
    if rem:
        pltpu.make_async_copy(
            den_v.at[pl.ds(0, rem)], acc.at[pl.ds(0, rem)], lsemA).wait()
    plsc.subcore_barrier()

    def fire_l(i, sidx_v, cx_v, cg_v, lsem):
        pltpu.async_copy(
            sidx_hbm.at[pl.ds(ibase + i * CHUNK3, CHUNK3)], sidx_v, lsem)
        pltpu.async_copy(
            cx_hbm.at[pl.ds(cbase + i * CHUNK3, CHUNK3)], cx_v, lsem)
        pltpu.async_copy(
            cg_hbm.at[pl.ds(cbase + i * CHUNK3, CHUNK3)], cg_v, lsem)

    def wait_l(sidx_v, cx_v, cg_v, lsem):
        pltpu.make_async_copy(
            sidx_hbm.at[pl.ds(0, CHUNK3)], sidx_v, lsem).wait()
        pltpu.make_async_copy(cx_hbm.at[pl.ds(0, CHUNK3)], cx_v, lsem).wait()
        pltpu.make_async_copy(cg_hbm.at[pl.ds(0, CHUNK3)], cg_v, lsem).wait()

    def den_comp(sidx_v, cg_v, nk):
        for k in range(nk):
            row16 = lax.broadcasted_iota(jnp.int32, (16,), 0) + k * 16
            col16 = jnp.zeros((16,), jnp.int32)
            g16 = plsc.load_gather(cg_v, [row16, col16])
            s16 = sidx_v[pl.ds(k * 16, 16)]
            plsc.addupdate_scatter(
                den_v, [lax.shift_right_logical(s16, 7),
                        lax.bitwise_and(s16, 127)], g16)

    def fire_s(sidx_v, cx_v, ssem):
        pltpu.async_copy(cx_v, acc.at[sidx_v], ssem, add=True)

    def wait_s(cx_v, ssem):
        pltpu.make_async_copy(cx_v, acc.at[pl.ds(0, CHUNK3)], ssem).wait()

    fire_l(0, sidxA, cxA, cgA, lsemA)

    def body(j, _):
        ib = 2 * j + 1
        fire_l(ib, sidxB, cxB, cgB, lsemB)
        wait_l(sidxA, cxA, cgA, lsemA)
        den_comp(sidxA, cgA, CHUNK3 // 16)
        fire_s(sidxA, cxA, ssemA)
        wait_s(cxA, ssemA)

        @pl.when(ib + 1 < nch)
        def _():
            fire_l(ib + 1, sidxA, cxA, cgA, lsemA)

        wait_l(sidxB, cxB, cgB, lsemB)
        den_comp(sidxB, cgB, CHUNK3 // 16)
        fire_s(sidxB, cxB, ssemB)
        wait_s(cxB, ssemB)
        return 0

    lax.fori_loop(0, nch // 2, body, 0)
    if tail:
        tb_i = ibase + nch * CHUNK3
        tb_c = cbase + nch * CHUNK3
        pltpu.sync_copy(sidx_hbm.at[pl.ds(tb_i, 16)], sidxT)
        pltpu.sync_copy(cx_hbm.at[pl.ds(tb_c, 16)], cxT)
        pltpu.sync_copy(cg_hbm.at[pl.ds(tb_c, 16)], cgT)
        den_comp(sidxT, cgT, 1)
        pltpu.sync_copy(cxT, acc.at[sidxT], add=True)
    plsc.subcore_barrier()
    pltpu.sync_copy(den_v, accg.at[rowid_v], add=True)
    plsc.subcore_barrier()

    row0 = sid * RPS
    pltpu.sync_copy(acc.at[pl.ds(row0, RPS)],
                    partx_out.at[pl.ds(cid * NP + row0, RPS)])

    @pl.when(sid == 0)
    def _():
        pltpu.sync_copy(accg, partg_out.at[pl.ds(cid * DRO, DRO)])


@functools.cache
def _make_sc_scatter(pw, off0):
    mesh = plsc.VectorSubcoreMesh(core_axis_name="c", subcore_axis_name="s")
    return functools.partial(
        pl.kernel,
        out_type=(
            jax.ShapeDtypeStruct((2 * NP, D), jnp.float32),
            jax.ShapeDtypeStruct((NC * DRO, 128), jnp.float32),
        ),
        mesh=mesh,
        scratch_types=[
            pltpu.VMEM((CHUNK3,), jnp.int32),
            pltpu.VMEM((CHUNK3, D), jnp.float32),
            pltpu.VMEM((CHUNK3, 8), jnp.float32),
            pltpu.VMEM((CHUNK3,), jnp.int32),
            pltpu.VMEM((CHUNK3, D), jnp.float32),
            pltpu.VMEM((CHUNK3, 8), jnp.float32),
            pltpu.VMEM((16,), jnp.int32),
            pltpu.VMEM((16, D), jnp.float32),
            pltpu.VMEM((16, 8), jnp.float32),
            pltpu.VMEM((DRO, 128), jnp.float32),
            pltpu.VMEM((DRO,), jnp.int32),
            pltpu.VMEM_SHARED((NP, D), jnp.float32),
            pltpu.VMEM_SHARED((DRO, 128), jnp.float32),
            pltpu.SemaphoreType.DMA,
            pltpu.SemaphoreType.DMA,
            pltpu.SemaphoreType.DMA,
            pltpu.SemaphoreType.DMA,
        ],
        compiler_params=pltpu.CompilerParams(needs_layout_passes=False),
    )(functools.partial(_sc_scatter_body, pw, off0))


def _tc_epi_body(x0_ref, x1_ref, x2_ref, x3_ref, den_ref, feat_ref, out_ref):
    num = (x0_ref[...] + x1_ref[...]) + (x2_ref[...] + x3_ref[...])
    out_ref[...] = num / (den_ref[...] + 1e-10) + feat_ref[...]


def _tc_epilogue(x0, x1, x2, x3, den, feat):
    T = 400
    grid = (N // T,)
    return pl.pallas_call(
        _tc_epi_body,
        grid=grid,
        in_specs=[
            pl.BlockSpec((T, D), lambda i: (i, 0)),
            pl.BlockSpec((T, D), lambda i: (i, 0)),
            pl.BlockSpec((T, D), lambda i: (i, 0)),
            pl.BlockSpec((T, D), lambda i: (i, 0)),
            pl.BlockSpec((T, 1), lambda i: (i, 0)),
            pl.BlockSpec((T, D), lambda i: (i, 0)),
        ],
        out_specs=pl.BlockSpec((T, D), lambda i: (i, 0)),
        out_shape=jax.ShapeDtypeStruct((N, D), jnp.float32),
    )(x0, x1, x2, x3, den, feat)


def kernel(node_weights, node_prev_features, self_idx, neighbor_idx, pow_p,
           gW0, gb0, gW1, gb1, mW0, mb0, mW1, mb1):
    feat = node_prev_features
    wpad = jnp.concatenate(
        [node_weights.reshape(N),
         jnp.zeros((DRO * 128 - N,), jnp.float32)]).reshape(DRO, 128)

    w0 = jnp.concatenate([gW0, mW0], axis=1)
    b0 = jnp.broadcast_to(jnp.concatenate([gb0, mb0])[None, :], (8, 2 * H))
    w1 = jnp.zeros((2 * H, YW), jnp.float32)
    w1 = w1.at[H:2 * H, 0:D].set(mW1)
    w1 = w1.at[0:H, D].set(gW1[:, 0])
    b1 = jnp.zeros((YW,), jnp.float32)
    b1 = b1.at[0:D].set(mb1)
    b1 = b1.at[D].set(gb1[0])
    b1 = jnp.broadcast_to(b1[None, :], (8, YW))
    p = jnp.broadcast_to(pow_p.reshape(1, 1), (8, 128))

    pxs, pgs = [], []
    prev = None
    for pw, off0 in PHASES:
        ms_p, mn_p, wn_p = _make_sc_gather(pw, off0)(
            feat, wpad, self_idx, neighbor_idx)
        cx_p, cg_p = _tc_mlp(ms_p, mn_p, wn_p.reshape(NW * pw, 1),
                             w0, b0, w1, b1, p)
        if prev is not None:
            px_q, pg_q = _make_sc_scatter(*prev[0])(prev[1], prev[2],
                                                    self_idx)
            pxs.append(px_q)
            pgs.append(pg_q)
        prev = ((pw, off0), cx_p, cg_p)
    px_q, pg_q = _make_sc_scatter(*prev[0])(prev[1], prev[2], self_idx)
    pxs.append(px_q)
    pgs.append(pg_q)
    den = sum(pg[0:DRO] + pg[DRO:2 * DRO] for pg in pgs)
    den = den.reshape(DRO * 128)[0:N]
    num01 = pxs[0][0:N] + pxs[1][0:N]
    num11 = pxs[0][NP:NP + N] + pxs[1][NP:NP + N]
    return _tc_epilogue(num01, num11,
                        pxs[2][0:N], pxs[2][NP:NP + N],
                        den.reshape(N, 1), feat)
